# baseline (speedup 1.0000x reference)
import functools

import jax
import jax.numpy as jnp
from jax import lax
from jax.experimental import pallas as pl
from jax.experimental.pallas import tpu as pltpu
from jax.experimental.pallas import tpu_sc as plsc

BATCH = 4096
HIST = 200
EMBED = 64
VOCAB = 1000000
NC = 2
NS = 16
NW = NC * NS
LANES = 128
TT = HIST // 8
FULL_TILES = VOCAB // LANES
TILES_W = FULL_TILES // NW
EXTRA_TILES = FULL_TILES - TILES_W * NW
TAIL_IDS = VOCAB - FULL_TILES * LANES


def _detile_body(tableT_hbm, tail_hbm, out_hbm,
                 b0, b1, bt0, bt1, tb,
                 semi0, semi1, semo0, semo1):
  cid = lax.axis_index("c")
  sid = lax.axis_index("s")
  wid = sid * NC + cid
  base = wid * TILES_W

  bufs = (b0, b1)
  bufts = (bt0, bt1)
  semi = (semi0, semi1)
  semo = (semo0, semo1)

  def fire_in(tc, s):
    off = pl.multiple_of(tc * LANES, LANES)
    pltpu.async_copy(tableT_hbm.at[:, pl.ds(off, LANES)], bufs[s], semi[s])

  def wait_in(s):
    pltpu.make_async_copy(tableT_hbm.at[:, pl.ds(0, LANES)], bufs[s],
                          semi[s]).wait()

  def wait_out(s):
    pltpu.make_async_copy(bufts[s], out_hbm.at[pl.ds(0, EMBED)],
                          semo[s]).wait()

  iota = lax.iota(jnp.int32, 16)
  idvecs = [16 * c + iota for c in range(LANES // 16)]
  rowvecs = [(16 * c + iota) // 2 for c in range(LANES // 16)]
  parvecs = [(16 * c + iota) % 2 * 64 for c in range(LANES // 16)]

  def transpose(s):
    @plsc.parallel_loop(0, EMBED, unroll=8)
    def _(k0):
      kv = (k0 + iota) & (EMBED - 1)
      for c in range(LANES // 16):
        x = plsc.load_gather(bufs[s], [kv, idvecs[c]])
        plsc.store_scatter(bufts[s], [rowvecs[c], parvecs[c] + kv], x)

  def fire_out(tc, s):
    off = pl.multiple_of(tc * (LANES // 2), LANES // 2)
    pltpu.async_copy(bufts[s], out_hbm.at[pl.ds(off, EMBED)], semo[s])

  fire_in(base, 0)
  fire_in(base + 1, 1)

  def loop(i, carry):
    for s in range(2):
      t = 2 * i + s
      tc = base + t
      wait_in(s)

      @pl.when(t >= 2)
      def _():
        wait_out(s)

      transpose(s)
      fire_out(tc, s)

      @pl.when(t + 2 < TILES_W)
      def _():
        fire_in(tc + 2, s)
    return carry

  lax.fori_loop(0, TILES_W // 2, loop, 0)
  wait_out(0)
  wait_out(1)

  @pl.when(wid < EXTRA_TILES)
  def _():
    tc = NW * TILES_W + wid
    fire_in(tc, 0)
    wait_in(0)
    transpose(0)
    fire_out(tc, 0)
    wait_out(0)

  @pl.when(wid == NW - 1)
  def _():
    pltpu.sync_copy(tail_hbm, tb)
    pltpu.sync_copy(tb, out_hbm.at[pl.ds(FULL_TILES * (LANES // 2),
                                         TAIL_IDS // 2)])


@functools.lru_cache(maxsize=1)
def _build_detile():
  return functools.partial(
      pl.kernel,
      out_type=jax.ShapeDtypeStruct((VOCAB // 2, 2 * EMBED), jnp.float32),
      mesh=plsc.VectorSubcoreMesh(core_axis_name="c", subcore_axis_name="s"),
      scratch_types=[
          pltpu.VMEM((EMBED, LANES), jnp.float32),
          pltpu.VMEM((EMBED, LANES), jnp.float32),
          pltpu.VMEM((EMBED, LANES), jnp.float32),
          pltpu.VMEM((EMBED, LANES), jnp.float32),
          pltpu.VMEM((TAIL_IDS // 2, 2 * EMBED), jnp.float32),
          pltpu.SemaphoreType.DMA,
          pltpu.SemaphoreType.DMA,
          pltpu.SemaphoreType.DMA,
          pltpu.SemaphoreType.DMA,
      ],
      compiler_params=pltpu.CompilerParams(use_tc_tiling_on_sc=True,
                                           needs_layout_passes=False),
  )(_detile_body)


def _body(idx4_hbm, mask4_hbm, table_hbm, tw_hbm, out_hbm,
          idx_v, mask_v, rows0, rows1, rows2, rows3, twb0, twb1, twb2, twb3,
          out_acc, out_t,
          sem_r0, sem_r1, sem_r2, sem_r3, sem_t0, sem_t1, sem_t2, sem_t3):
  cid = lax.axis_index("c")
  sid = lax.axis_index("s")
  wid = sid * NC + cid

  pltpu.sync_copy(idx4_hbm.at[:, wid], idx_v)
  pltpu.sync_copy(mask4_hbm.at[:, wid], mask_v)

  rows_bufs = (rows0, rows1, rows2, rows3)
  tw_bufs = (twb0, twb1, twb2, twb3)
  sem_r = (sem_r0, sem_r1, sem_r2, sem_r3)
  sem_t = (sem_t0, sem_t1, sem_t2, sem_t3)

  z = jnp.zeros((16,), jnp.float32)

  def zero_body(i, carry):
    for k in range(EMBED // 16):
      out_acc[i, pl.ds(16 * k, 16)] = z
    return carry

  lax.fori_loop(0, LANES, zero_body, 0)

  def fire(tr, r, b):
    idxr = idx_v.at[tr, r]
    pltpu.async_copy(table_hbm.at[idxr], rows_bufs[b], sem_r[b])
    pltpu.async_copy(tw_hbm.at[idxr], tw_bufs[b], sem_t[b])

  def wait(b):
    pltpu.make_async_copy(table_hbm.at[pl.ds(0, LANES)], rows_bufs[b],
                          sem_r[b]).wait()
    pltpu.make_async_copy(tw_hbm.at[pl.ds(0, LANES)], tw_bufs[b],
                          sem_t[b]).wait()

  for t0 in range(4):
    fire(0, t0, t0)

  def outer(tr, carry):
    for r0 in range(0, 8, 2):
      b0 = r0 % 4
      b1 = (r0 + 1) % 4
      wait(b0)
      wait(b1)
      rows_a = rows_bufs[b0]
      rows_b = rows_bufs[b1]
      twa = tw_bufs[b0]
      twc = tw_bufs[b1]

      def chunk_body(c, carry2):
        ma = mask_v[tr, r0, pl.ds(16 * c, 16)]
        mb = mask_v[tr, r0 + 1, pl.ds(16 * c, 16)]
        ta = twa[pl.ds(16 * c, 16)]
        tb2 = twc[pl.ds(16 * c, 16)]
        wva = ma * ma * ta
        wvb = mb * mb * tb2
        for u in range(16):
          i = 16 * c + u
          wa = wva[u]
          wb = wvb[u]
          for k in range(EMBED // 16):
            sl = pl.ds(16 * k, 16)
            plsc.addupdate(out_acc.at[i, sl],
                           rows_a[i, sl] * wa + rows_b[i, sl] * wb)
        return carry2

      lax.fori_loop(0, LANES // 16, chunk_body, 0)

      for dr in range(2):
        r = r0 + dr
        b = r % 4
        if r < 4:
          fire(tr, r + 4, b)
        else:
          @pl.when(tr + 1 < TT)
          def _():
            fire(tr + 1, r - 4, b)
    return carry

  lax.fori_loop(0, TT, outer, 0)

  iota = lax.iota(jnp.int32, 16)
  for c in range(LANES // 16):
    bidx = 16 * c + iota
    for k in range(EMBED):
      kidx = jnp.full((16,), k, jnp.int32)
      out_t[k // 8, k % 8, pl.ds(16 * c, 16)] = plsc.load_gather(
          out_acc, [bidx, kidx])

  pltpu.sync_copy(out_t, out_hbm.at[:, wid])


@functools.lru_cache(maxsize=1)
def _build():
  return functools.partial(
      pl.kernel,
      out_type=jax.ShapeDtypeStruct((EMBED // 8, NW, 8, LANES), jnp.float32),
      mesh=plsc.VectorSubcoreMesh(core_axis_name="c", subcore_axis_name="s"),
      scratch_types=[
          pltpu.VMEM((TT, 8, LANES), jnp.int32),
          pltpu.VMEM((TT, 8, LANES), jnp.float32),
          pltpu.VMEM((LANES, EMBED), jnp.float32),
          pltpu.VMEM((LANES, EMBED), jnp.float32),
          pltpu.VMEM((LANES, EMBED), jnp.float32),
          pltpu.VMEM((LANES, EMBED), jnp.float32),
          pltpu.VMEM((LANES,), jnp.float32),
          pltpu.VMEM((LANES,), jnp.float32),
          pltpu.VMEM((LANES,), jnp.float32),
          pltpu.VMEM((LANES,), jnp.float32),
          pltpu.VMEM((LANES, EMBED), jnp.float32),
          pltpu.VMEM((EMBED // 8, 8, LANES), jnp.float32),
          pltpu.SemaphoreType.DMA,
          pltpu.SemaphoreType.DMA,
          pltpu.SemaphoreType.DMA,
          pltpu.SemaphoreType.DMA,
          pltpu.SemaphoreType.DMA,
          pltpu.SemaphoreType.DMA,
          pltpu.SemaphoreType.DMA,
          pltpu.SemaphoreType.DMA,
      ],
      compiler_params=pltpu.CompilerParams(use_tc_tiling_on_sc=False,
                                           needs_layout_passes=False),
  )(_body)


def kernel(idxs, mask, table, token_weights):
  idx4 = idxs.astype(jnp.int32).reshape(32, 128, 25, 8).transpose(2, 0, 3, 1)
  mask4 = mask.reshape(32, 128, 25, 8).transpose(2, 0, 3, 1)
  tail = table[FULL_TILES * LANES:].reshape(TAIL_IDS // 2, 2 * EMBED)
  tableL2 = _build_detile()(table.T, tail)
  tableL = tableL2.reshape(VOCAB, EMBED)
  out4 = _build()(idx4, mask4, tableL, token_weights)
  return out4.transpose(1, 3, 0, 2).reshape(BATCH, EMBED)


# speedup vs baseline: 1.0520x; 1.0520x over previous
import functools

import jax
import jax.numpy as jnp
from jax import lax
from jax.experimental import pallas as pl
from jax.experimental.pallas import tpu as pltpu
from jax.experimental.pallas import tpu_sc as plsc

BATCH = 4096
HIST = 200
EMBED = 64
VOCAB = 1000000
NC = 2
NS = 16
NW = NC * NS
LANES = 128
TT = HIST // 8
FULL_TILES = VOCAB // LANES
TILES_W = FULL_TILES // NW
EXTRA_TILES = FULL_TILES - TILES_W * NW
TAIL_IDS = VOCAB - FULL_TILES * LANES


def _detile_body(tableT_hbm, tail_hbm, out_hbm,
                 b0, b1, bt0, bt1, tb,
                 semi0, semi1, semo0, semo1):
  cid = lax.axis_index("c")
  sid = lax.axis_index("s")
  wid = sid * NC + cid
  base = wid * TILES_W

  bufs = (b0, b1)
  bufts = (bt0, bt1)
  semi = (semi0, semi1)
  semo = (semo0, semo1)

  def fire_in(tc, s):
    off = pl.multiple_of(tc * LANES, LANES)
    pltpu.async_copy(tableT_hbm.at[:, pl.ds(off, LANES)], bufs[s], semi[s])

  def wait_in(s):
    pltpu.make_async_copy(tableT_hbm.at[:, pl.ds(0, LANES)], bufs[s],
                          semi[s]).wait()

  def wait_out(s):
    pltpu.make_async_copy(bufts[s], out_hbm.at[pl.ds(0, EMBED)],
                          semo[s]).wait()

  iota = lax.iota(jnp.int32, 16)
  idvecs = [16 * c + iota for c in range(LANES // 16)]
  rowvecs = [(16 * c + iota) // 2 for c in range(LANES // 16)]
  parvecs = [(16 * c + iota) % 2 * 64 for c in range(LANES // 16)]

  def transpose(s):
    @plsc.parallel_loop(0, EMBED, unroll=4)
    def _(k0):
      kv = (k0 + iota) & (EMBED - 1)
      for c in range(LANES // 16):
        x = plsc.load_gather(bufs[s], [kv, idvecs[c]])
        plsc.store_scatter(bufts[s], [rowvecs[c], parvecs[c] + kv], x)

  def fire_out(tc, s):
    off = pl.multiple_of(tc * (LANES // 2), LANES // 2)
    pltpu.async_copy(bufts[s], out_hbm.at[pl.ds(off, EMBED)], semo[s])

  fire_in(base, 0)
  fire_in(base + 1, 1)

  def loop(i, carry):
    for s in range(2):
      t = 2 * i + s
      tc = base + t
      wait_in(s)

      @pl.when(t >= 2)
      def _():
        wait_out(s)

      transpose(s)
      fire_out(tc, s)

      @pl.when(t + 2 < TILES_W)
      def _():
        fire_in(tc + 2, s)
    return carry

  lax.fori_loop(0, TILES_W // 2, loop, 0)
  wait_out(0)
  wait_out(1)

  @pl.when(wid < EXTRA_TILES)
  def _():
    tc = NW * TILES_W + wid
    fire_in(tc, 0)
    wait_in(0)
    transpose(0)
    fire_out(tc, 0)
    wait_out(0)

  @pl.when(wid == NW - 1)
  def _():
    pltpu.sync_copy(tail_hbm, tb)
    pltpu.sync_copy(tb, out_hbm.at[pl.ds(FULL_TILES * (LANES // 2),
                                         TAIL_IDS // 2)])


@functools.lru_cache(maxsize=1)
def _build_detile():
  return functools.partial(
      pl.kernel,
      out_type=jax.ShapeDtypeStruct((VOCAB // 2, 2 * EMBED), jnp.float32),
      mesh=plsc.VectorSubcoreMesh(core_axis_name="c", subcore_axis_name="s"),
      scratch_types=[
          pltpu.VMEM((EMBED, LANES), jnp.float32),
          pltpu.VMEM((EMBED, LANES), jnp.float32),
          pltpu.VMEM((EMBED, LANES), jnp.float32),
          pltpu.VMEM((EMBED, LANES), jnp.float32),
          pltpu.VMEM((TAIL_IDS // 2, 2 * EMBED), jnp.float32),
          pltpu.SemaphoreType.DMA,
          pltpu.SemaphoreType.DMA,
          pltpu.SemaphoreType.DMA,
          pltpu.SemaphoreType.DMA,
      ],
      compiler_params=pltpu.CompilerParams(use_tc_tiling_on_sc=True,
                                           needs_layout_passes=False),
  )(_detile_body)


def _body(idx4_hbm, mask4_hbm, table_hbm, tw_hbm, out_hbm,
          idx_v, mask_v, rows0, rows1, rows2, rows3, twb0, twb1, twb2, twb3,
          out_acc, out_t,
          sem_r0, sem_r1, sem_r2, sem_r3, sem_t0, sem_t1, sem_t2, sem_t3):
  cid = lax.axis_index("c")
  sid = lax.axis_index("s")
  wid = sid * NC + cid

  pltpu.sync_copy(idx4_hbm.at[:, wid], idx_v)
  pltpu.sync_copy(mask4_hbm.at[:, wid], mask_v)

  rows_bufs = (rows0, rows1, rows2, rows3)
  tw_bufs = (twb0, twb1, twb2, twb3)
  sem_r = (sem_r0, sem_r1, sem_r2, sem_r3)
  sem_t = (sem_t0, sem_t1, sem_t2, sem_t3)

  z = jnp.zeros((16,), jnp.float32)

  def zero_body(i, carry):
    for k in range(EMBED // 16):
      out_acc[i, pl.ds(16 * k, 16)] = z
    return carry

  lax.fori_loop(0, LANES, zero_body, 0)

  def fire(tr, r, b):
    idxr = idx_v.at[tr, r]
    pltpu.async_copy(table_hbm.at[idxr], rows_bufs[b], sem_r[b])
    pltpu.async_copy(tw_hbm.at[idxr], tw_bufs[b], sem_t[b])

  def wait(b):
    pltpu.make_async_copy(table_hbm.at[pl.ds(0, LANES)], rows_bufs[b],
                          sem_r[b]).wait()
    pltpu.make_async_copy(tw_hbm.at[pl.ds(0, LANES)], tw_bufs[b],
                          sem_t[b]).wait()

  for t0 in range(4):
    fire(0, t0, t0)

  def outer(tr, carry):
    for r0 in range(0, 8, 2):
      b0 = r0 % 4
      b1 = (r0 + 1) % 4
      wait(b0)
      wait(b1)
      rows_a = rows_bufs[b0]
      rows_b = rows_bufs[b1]
      twa = tw_bufs[b0]
      twc = tw_bufs[b1]

      def chunk_body(c, carry2):
        ma = mask_v[tr, r0, pl.ds(16 * c, 16)]
        mb = mask_v[tr, r0 + 1, pl.ds(16 * c, 16)]
        ta = twa[pl.ds(16 * c, 16)]
        tb2 = twc[pl.ds(16 * c, 16)]
        wva = ma * ma * ta
        wvb = mb * mb * tb2
        for u in range(16):
          i = 16 * c + u
          wa = wva[u]
          wb = wvb[u]
          for k in range(EMBED // 16):
            sl = pl.ds(16 * k, 16)
            plsc.addupdate(out_acc.at[i, sl],
                           rows_a[i, sl] * wa + rows_b[i, sl] * wb)
        return carry2

      lax.fori_loop(0, LANES // 16, chunk_body, 0)

      for dr in range(2):
        r = r0 + dr
        b = r % 4
        if r < 4:
          fire(tr, r + 4, b)
        else:
          @pl.when(tr + 1 < TT)
          def _():
            fire(tr + 1, r - 4, b)
    return carry

  lax.fori_loop(0, TT, outer, 0)

  iota = lax.iota(jnp.int32, 16)
  for c in range(LANES // 16):
    bidx = 16 * c + iota
    for k in range(EMBED):
      kidx = jnp.full((16,), k, jnp.int32)
      out_t[k // 8, k % 8, pl.ds(16 * c, 16)] = plsc.load_gather(
          out_acc, [bidx, kidx])

  pltpu.sync_copy(out_t, out_hbm.at[:, wid])


@functools.lru_cache(maxsize=1)
def _build():
  return functools.partial(
      pl.kernel,
      out_type=jax.ShapeDtypeStruct((EMBED // 8, NW, 8, LANES), jnp.float32),
      mesh=plsc.VectorSubcoreMesh(core_axis_name="c", subcore_axis_name="s"),
      scratch_types=[
          pltpu.VMEM((TT, 8, LANES), jnp.int32),
          pltpu.VMEM((TT, 8, LANES), jnp.float32),
          pltpu.VMEM((LANES, EMBED), jnp.float32),
          pltpu.VMEM((LANES, EMBED), jnp.float32),
          pltpu.VMEM((LANES, EMBED), jnp.float32),
          pltpu.VMEM((LANES, EMBED), jnp.float32),
          pltpu.VMEM((LANES,), jnp.float32),
          pltpu.VMEM((LANES,), jnp.float32),
          pltpu.VMEM((LANES,), jnp.float32),
          pltpu.VMEM((LANES,), jnp.float32),
          pltpu.VMEM((LANES, EMBED), jnp.float32),
          pltpu.VMEM((EMBED // 8, 8, LANES), jnp.float32),
          pltpu.SemaphoreType.DMA,
          pltpu.SemaphoreType.DMA,
          pltpu.SemaphoreType.DMA,
          pltpu.SemaphoreType.DMA,
          pltpu.SemaphoreType.DMA,
          pltpu.SemaphoreType.DMA,
          pltpu.SemaphoreType.DMA,
          pltpu.SemaphoreType.DMA,
      ],
      compiler_params=pltpu.CompilerParams(use_tc_tiling_on_sc=False,
                                           needs_layout_passes=False),
  )(_body)


def kernel(idxs, mask, table, token_weights):
  idx4 = idxs.astype(jnp.int32).reshape(32, 128, 25, 8).transpose(2, 0, 3, 1)
  mask4 = mask.reshape(32, 128, 25, 8).transpose(2, 0, 3, 1)
  tail = table[FULL_TILES * LANES:].reshape(TAIL_IDS // 2, 2 * EMBED)
  tableL2 = _build_detile()(table.T, tail)
  tableL = tableL2.reshape(VOCAB, EMBED)
  out4 = _build()(idx4, mask4, tableL, token_weights)
  return out4.transpose(1, 3, 0, 2).reshape(BATCH, EMBED)
